# Initial kernel scaffold; baseline (speedup 1.0000x reference)
import functools

import jax
import jax.numpy as jnp
from jax import lax
from jax.experimental import pallas as pl
from jax.experimental.pallas import tpu as pltpu
from jax.experimental.pallas import tpu_sc as plsc

NUM_CORES = 2
NUM_SUBCORES = 16
K = 80
BN_EPS = 1e-5


def _make_sc_aggregate(n, d, e):
    nw = NUM_CORES * NUM_SUBCORES
    assert e % nw == 0 and (e // nw) % K == 0 and n % NUM_SUBCORES == 0
    epw = e // nw
    n_chunks = epw // K
    rpt = n // NUM_SUBCORES

    mesh = plsc.VectorSubcoreMesh(
        core_axis_name="c", subcore_axis_name="s",
        num_cores=NUM_CORES, num_subcores=NUM_SUBCORES)

    @functools.partial(
        pl.kernel,
        out_type=jax.ShapeDtypeStruct((NUM_CORES, n, d), jnp.float32),
        mesh=mesh,
        scratch_types=[
            pltpu.VMEM_SHARED((n, d), jnp.float32),
            pltpu.VMEM((K,), jnp.int32),
            pltpu.VMEM((K,), jnp.int32),
            pltpu.VMEM((K, d), jnp.float32),
            pltpu.SemaphoreType.DMA,
        ],
    )
    def agg(h_hbm, src_hbm, dst_hbm, part_hbm, accum, src_v, dst_v, rows_v, sem):
        c = lax.axis_index("c")
        s = lax.axis_index("s")

        row0 = s * rpt

        @pl.when(c == 0)
        def _():
            pltpu.sync_copy(h_hbm.at[pl.ds(row0, rpt)], accum.at[pl.ds(row0, rpt)])

        @pl.when(c != 0)
        def _():
            zv = jnp.zeros((16,), jnp.float32)

            def zero_row(i, _):
                rows_v[0, pl.ds(i * 16, 16)] = zv
                return 0
            lax.fori_loop(0, d // 16, zero_row, 0)

            def zero_body(i, _):
                pltpu.sync_copy(rows_v.at[0], accum.at[row0 + i])
                return 0
            lax.fori_loop(0, rpt, zero_body, 0)

        plsc.subcore_barrier()

        e_base = (c * NUM_SUBCORES + s) * epw

        def chunk_body(j, _):
            off = e_base + j * K
            pltpu.sync_copy(src_hbm.at[pl.ds(off, K)], src_v)
            pltpu.sync_copy(dst_hbm.at[pl.ds(off, K)], dst_v)
            pltpu.async_copy(h_hbm.at[src_v], rows_v, sem).wait()
            pltpu.sync_copy(rows_v, accum.at[dst_v], add=True)
            return 0
        lax.fori_loop(0, n_chunks, chunk_body, 0)

        plsc.subcore_barrier()

        pltpu.sync_copy(accum.at[pl.ds(row0, rpt)], part_hbm.at[c, pl.ds(row0, rpt)])

    return agg


def _mlp_block(p0_ref, p1_ref, w1_ref, b1_ref, w2_ref, b2_ref, o_ref):
    z = p0_ref[...] + p1_ref[...]
    h1 = jnp.dot(z, w1_ref[...], preferred_element_type=jnp.float32) + b1_ref[...]
    h1 = jnp.maximum(h1, 0.0)
    h2 = jnp.dot(h1, w2_ref[...], preferred_element_type=jnp.float32) + b2_ref[...]
    o_ref[...] = jnp.maximum(h2, 0.0)


def _make_tc_mlp(n, d, h_dim, block_rows):
    assert n % block_rows == 0
    grid = (n // block_rows,)
    row_spec = pl.BlockSpec((block_rows, d), lambda i: (i, 0))
    full = lambda r, c0: pl.BlockSpec((r, c0), lambda i: (0, 0))
    return pl.pallas_call(
        _mlp_block,
        grid=grid,
        in_specs=[row_spec, row_spec,
                  full(d, h_dim), full(1, h_dim),
                  full(h_dim, h_dim), full(1, h_dim)],
        out_specs=pl.BlockSpec((block_rows, h_dim), lambda i: (i, 0)),
        out_shape=jax.ShapeDtypeStruct((n, h_dim), jnp.float32),
    )


def kernel(x, edge_index, params):
    n, d = x.shape
    e = edge_index.shape[1]
    src = edge_index[0].astype(jnp.int32)
    dst = edge_index[1].astype(jnp.int32)

    agg = _make_sc_aggregate(n, d, e)
    mlp = _make_tc_mlp(n, d, d, 2000)

    inv = 1.0 / jnp.sqrt(1.0 + BN_EPS)
    h = x
    for i in range(3):
        s1 = params[f"g1_{i}"] * inv
        w1f = params[f"W1_{i}"] * s1[None, :]
        b1f = (params[f"b1_{i}"] * s1 + params[f"be1_{i}"])[None, :]
        s2 = params[f"g2_{i}"] * inv
        w2f = params[f"W2_{i}"] * s2[None, :]
        b2f = (params[f"b2_{i}"] * s2 + params[f"be2_{i}"])[None, :]
        part = agg(h, src, dst)
        h = mlp(part[0], part[1], w1f, b1f, w2f, b2f)
    return h


# speedup vs baseline: 4.6181x; 4.6181x over previous
import functools

import jax
import jax.numpy as jnp
from jax import lax
from jax.experimental import pallas as pl
from jax.experimental.pallas import tpu as pltpu
from jax.experimental.pallas import tpu_sc as plsc

NUM_CORES = 2
NUM_SUBCORES = 16
K = 80
BN_EPS = 1e-5


def _make_sc_aggregate(n, d, e):
    nw = NUM_CORES * NUM_SUBCORES
    assert e % nw == 0 and (e // nw) % K == 0 and n % NUM_SUBCORES == 0
    epw = e // nw
    n_chunks = epw // K
    rpt = n // NUM_SUBCORES

    mesh = plsc.VectorSubcoreMesh(
        core_axis_name="c", subcore_axis_name="s",
        num_cores=NUM_CORES, num_subcores=NUM_SUBCORES)

    @functools.partial(
        pl.kernel,
        out_type=jax.ShapeDtypeStruct((NUM_CORES, n, d), jnp.float32),
        mesh=mesh,
        scratch_types=[
            pltpu.VMEM_SHARED((n, d), jnp.float32),
            pltpu.VMEM((K,), jnp.int32),
            pltpu.VMEM((K,), jnp.int32),
            pltpu.VMEM((K, d), jnp.float32),
            pltpu.SemaphoreType.DMA,
        ],
        compiler_params=pltpu.CompilerParams(use_tc_tiling_on_sc=False),
    )
    def agg(h_hbm, src_hbm, dst_hbm, part_hbm, accum, src_v, dst_v, rows_v, sem):
        c = lax.axis_index("c")
        s = lax.axis_index("s")

        row0 = s * rpt

        @pl.when(c == 0)
        def _():
            pltpu.sync_copy(h_hbm.at[pl.ds(row0, rpt)], accum.at[pl.ds(row0, rpt)])

        @pl.when(c != 0)
        def _():
            zv = jnp.zeros((16,), jnp.float32)

            def zero_row(i, _):
                rows_v[0, pl.ds(i * 16, 16)] = zv
                return 0
            lax.fori_loop(0, d // 16, zero_row, 0)

            def zero_body(i, _):
                pltpu.sync_copy(rows_v.at[0], accum.at[row0 + i])
                return 0
            lax.fori_loop(0, rpt, zero_body, 0)

        plsc.subcore_barrier()

        e_base = (c * NUM_SUBCORES + s) * epw

        def chunk_body(j, _):
            off = e_base + j * K
            pltpu.sync_copy(src_hbm.at[pl.ds(off, K)], src_v)
            pltpu.sync_copy(dst_hbm.at[pl.ds(off, K)], dst_v)
            pltpu.async_copy(h_hbm.at[src_v], rows_v, sem).wait()
            pltpu.sync_copy(rows_v, accum.at[dst_v], add=True)
            return 0
        lax.fori_loop(0, n_chunks, chunk_body, 0)

        plsc.subcore_barrier()

        pltpu.sync_copy(accum.at[pl.ds(row0, rpt)], part_hbm.at[c, pl.ds(row0, rpt)])

    return agg


def _mlp_block(p0_ref, p1_ref, w1_ref, b1_ref, w2_ref, b2_ref, o_ref):
    z = p0_ref[...] + p1_ref[...]
    h1 = jnp.dot(z, w1_ref[...], preferred_element_type=jnp.float32) + b1_ref[...]
    h1 = jnp.maximum(h1, 0.0)
    h2 = jnp.dot(h1, w2_ref[...], preferred_element_type=jnp.float32) + b2_ref[...]
    o_ref[...] = jnp.maximum(h2, 0.0)


def _make_tc_mlp(n, d, h_dim, block_rows):
    assert n % block_rows == 0
    grid = (n // block_rows,)
    row_spec = pl.BlockSpec((block_rows, d), lambda i: (i, 0))
    full = lambda r, c0: pl.BlockSpec((r, c0), lambda i: (0, 0))
    return pl.pallas_call(
        _mlp_block,
        grid=grid,
        in_specs=[row_spec, row_spec,
                  full(d, h_dim), full(1, h_dim),
                  full(h_dim, h_dim), full(1, h_dim)],
        out_specs=pl.BlockSpec((block_rows, h_dim), lambda i: (i, 0)),
        out_shape=jax.ShapeDtypeStruct((n, h_dim), jnp.float32),
    )


def kernel(x, edge_index, params):
    n, d = x.shape
    e = edge_index.shape[1]
    src = edge_index[0].astype(jnp.int32)
    dst = edge_index[1].astype(jnp.int32)

    agg = _make_sc_aggregate(n, d, e)
    mlp = _make_tc_mlp(n, d, d, 2000)

    inv = 1.0 / jnp.sqrt(1.0 + BN_EPS)
    h = x
    for i in range(3):
        s1 = params[f"g1_{i}"] * inv
        w1f = params[f"W1_{i}"] * s1[None, :]
        b1f = (params[f"b1_{i}"] * s1 + params[f"be1_{i}"])[None, :]
        s2 = params[f"g2_{i}"] * inv
        w2f = params[f"W2_{i}"] * s2[None, :]
        b2f = (params[f"b2_{i}"] * s2 + params[f"be2_{i}"])[None, :]
        part = agg(h, src, dst)
        h = mlp(part[0], part[1], w1f, b1f, w2f, b2f)
    return h


# speedup vs baseline: 9.1972x; 1.9916x over previous
import functools

import jax
import jax.numpy as jnp
from jax import lax
from jax.experimental import pallas as pl
from jax.experimental.pallas import tpu as pltpu
from jax.experimental.pallas import tpu_sc as plsc

NUM_CORES = 2
NUM_SUBCORES = 16
K = 80
NBUF = 4
LOOK = 2
BN_EPS = 1e-5


def _make_sc_aggregate(n, d, e):
    nw = NUM_CORES * NUM_SUBCORES
    assert e % nw == 0 and (e // nw) % K == 0 and n % NUM_SUBCORES == 0
    epw = e // nw
    n_chunks = epw // K
    rpt = n // NUM_SUBCORES

    mesh = plsc.VectorSubcoreMesh(
        core_axis_name="c", subcore_axis_name="s",
        num_cores=NUM_CORES, num_subcores=NUM_SUBCORES)

    assert n_chunks > 2 * NBUF and LOOK < NBUF

    @functools.partial(
        pl.kernel,
        out_type=jax.ShapeDtypeStruct((NUM_CORES, n, d), jnp.float32),
        mesh=mesh,
        scratch_types=[
            pltpu.VMEM_SHARED((n, d), jnp.float32),
            pltpu.VMEM((NBUF, K), jnp.int32),
            pltpu.VMEM((NBUF, K), jnp.int32),
            pltpu.VMEM((NBUF, K, d), jnp.float32),
            pltpu.SemaphoreType.DMA((NBUF,)),
            pltpu.SemaphoreType.DMA((NBUF,)),
            pltpu.SemaphoreType.DMA((NBUF,)),
        ],
        compiler_params=pltpu.CompilerParams(use_tc_tiling_on_sc=False),
    )
    def agg(h_hbm, src_hbm, dst_hbm, part_hbm,
            accum, srci, dsti, rows, gsem, dsem, ssem):
        c = lax.axis_index("c")
        s = lax.axis_index("s")
        row0 = s * rpt
        e_base = (c * NUM_SUBCORES + s) * epw

        @pl.when(c == 0)
        def _():
            pltpu.sync_copy(h_hbm.at[pl.ds(row0, rpt)], accum.at[pl.ds(row0, rpt)])

        @pl.when(c != 0)
        def _():
            zv = jnp.zeros((16,), jnp.float32)

            def zero_vec(i, _):
                rows[0, i // 8, pl.ds((i % 8) * 16, 16)] = zv
                return 0
            lax.fori_loop(0, K * d // 16, zero_vec, 0)
            nfull = rpt // K
            for q in range(nfull):
                pltpu.sync_copy(rows.at[0], accum.at[pl.ds(row0 + q * K, K)])
            rem = rpt - nfull * K
            if rem:
                pltpu.sync_copy(rows.at[0].at[pl.ds(0, rem)],
                                accum.at[pl.ds(row0 + nfull * K, rem)])

        plsc.subcore_barrier()

        def issue_pref(j, b):
            pltpu.async_copy(src_hbm.at[pl.ds(e_base + j * K, K)],
                             srci.at[b], dsem.at[b])
            pltpu.async_copy(dst_hbm.at[pl.ds(e_base + j * K, K)],
                             dsti.at[b], dsem.at[b])

        def issue_gather(j, b):
            pltpu.async_copy(h_hbm.at[srci.at[b]],
                             rows.at[b], gsem.at[b])

        def wait_pref(b):
            pltpu.make_async_copy(src_hbm.at[pl.ds(0, K)], srci.at[b],
                                  dsem.at[b]).wait()
            pltpu.make_async_copy(dst_hbm.at[pl.ds(0, K)], dsti.at[b],
                                  dsem.at[b]).wait()

        def wait_gather(b):
            pltpu.make_async_copy(h_hbm.at[pl.ds(0, K)], rows.at[b],
                                  gsem.at[b]).wait()

        def issue_scatter(b):
            pltpu.async_copy(rows.at[b], accum.at[dsti.at[b]], ssem.at[b],
                             add=True)

        def wait_scatter(b):
            pltpu.make_async_copy(rows.at[b], accum.at[pl.ds(0, K)],
                                  ssem.at[b]).wait()

        issue_pref(0, 0)
        issue_pref(1, 1)
        wait_pref(0)
        issue_gather(0, 0)
        for j in range(2):
            wait_gather(j)
            issue_scatter(j)
            issue_pref(j + 2, j + 2)
            wait_pref(j + 1)
            issue_gather(j + 1, j + 1)

        def body(j, _):
            b = lax.rem(j, NBUF)
            bp = lax.rem(j + 2, NBUF)
            bg = lax.rem(j + 1, NBUF)
            wait_gather(b)
            issue_scatter(b)
            wait_scatter(bp)
            issue_pref(j + 2, bp)
            wait_pref(bg)
            issue_gather(j + 1, bg)
            return 0
        lax.fori_loop(2, n_chunks - 2, body, 0)

        j = n_chunks - 2
        wait_gather(j % NBUF)
        issue_scatter(j % NBUF)
        wait_pref((j + 1) % NBUF)
        issue_gather(j + 1, (j + 1) % NBUF)
        j = n_chunks - 1
        wait_gather(j % NBUF)
        issue_scatter(j % NBUF)
        for j in range(n_chunks - NBUF, n_chunks):
            wait_scatter(j % NBUF)

        plsc.subcore_barrier()

        pltpu.sync_copy(accum.at[pl.ds(row0, rpt)], part_hbm.at[c, pl.ds(row0, rpt)])

    return agg


def _mlp_block(p0_ref, p1_ref, w1_ref, b1_ref, w2_ref, b2_ref, o_ref):
    z = p0_ref[...] + p1_ref[...]
    h1 = jnp.dot(z, w1_ref[...], preferred_element_type=jnp.float32) + b1_ref[...]
    h1 = jnp.maximum(h1, 0.0)
    h2 = jnp.dot(h1, w2_ref[...], preferred_element_type=jnp.float32) + b2_ref[...]
    o_ref[...] = jnp.maximum(h2, 0.0)


def _make_tc_mlp(n, d, h_dim, block_rows):
    assert n % block_rows == 0
    grid = (n // block_rows,)
    row_spec = pl.BlockSpec((block_rows, d), lambda i: (i, 0))
    full = lambda r, c0: pl.BlockSpec((r, c0), lambda i: (0, 0))
    return pl.pallas_call(
        _mlp_block,
        grid=grid,
        in_specs=[row_spec, row_spec,
                  full(d, h_dim), full(1, h_dim),
                  full(h_dim, h_dim), full(1, h_dim)],
        out_specs=pl.BlockSpec((block_rows, h_dim), lambda i: (i, 0)),
        out_shape=jax.ShapeDtypeStruct((n, h_dim), jnp.float32),
    )


def kernel(x, edge_index, params):
    n, d = x.shape
    e = edge_index.shape[1]
    src = edge_index[0].astype(jnp.int32)
    dst = edge_index[1].astype(jnp.int32)

    agg = _make_sc_aggregate(n, d, e)
    mlp = _make_tc_mlp(n, d, d, 2000)

    inv = 1.0 / jnp.sqrt(1.0 + BN_EPS)
    h = x
    for i in range(3):
        s1 = params[f"g1_{i}"] * inv
        w1f = params[f"W1_{i}"] * s1[None, :]
        b1f = (params[f"b1_{i}"] * s1 + params[f"be1_{i}"])[None, :]
        s2 = params[f"g2_{i}"] * inv
        w2f = params[f"W2_{i}"] * s2[None, :]
        b2f = (params[f"b2_{i}"] * s2 + params[f"be2_{i}"])[None, :]
        part = agg(h, src, dst)
        h = mlp(part[0], part[1], w1f, b1f, w2f, b2f)
    return h


# speedup vs baseline: 13.3631x; 1.4530x over previous
import functools

import jax
import jax.numpy as jnp
from jax import lax
from jax.experimental import pallas as pl
from jax.experimental.pallas import tpu as pltpu
from jax.experimental.pallas import tpu_sc as plsc

NUM_CORES = 2
NUM_SUBCORES = 16
K = 80
NBUF = 4
LOOK = 2
BN_EPS = 1e-5


def _make_sc_aggregate(n, d, e):
    nw = NUM_CORES * NUM_SUBCORES
    assert e % nw == 0 and (e // nw) % K == 0 and n % NUM_SUBCORES == 0
    epw = e // nw
    n_chunks = epw // K
    rpt = n // NUM_SUBCORES

    mesh = plsc.VectorSubcoreMesh(
        core_axis_name="c", subcore_axis_name="s",
        num_cores=NUM_CORES, num_subcores=NUM_SUBCORES)

    assert n_chunks > 2 * NBUF and LOOK < NBUF

    @functools.partial(
        pl.kernel,
        out_type=jax.ShapeDtypeStruct((NUM_CORES, n, d), jnp.float32),
        mesh=mesh,
        scratch_types=[
            pltpu.VMEM_SHARED((n, d), jnp.float32),
            pltpu.VMEM((NBUF, K), jnp.int32),
            pltpu.VMEM((NBUF, K), jnp.int32),
            pltpu.VMEM((NBUF, K, d), jnp.float32),
            pltpu.SemaphoreType.DMA((NBUF,)),
            pltpu.SemaphoreType.DMA((NBUF,)),
            pltpu.SemaphoreType.DMA((NBUF,)),
        ],
        compiler_params=pltpu.CompilerParams(use_tc_tiling_on_sc=False),
    )
    def agg(h_hbm, src_hbm, dst_hbm, part_hbm,
            accum, srci, dsti, rows, gsem, dsem, ssem):
        c = lax.axis_index("c")
        s = lax.axis_index("s")
        row0 = s * rpt
        e_base = (c * NUM_SUBCORES + s) * epw

        @pl.when(c == 0)
        def _():
            pltpu.sync_copy(h_hbm.at[pl.ds(row0, rpt)], accum.at[pl.ds(row0, rpt)])

        @pl.when(c != 0)
        def _():
            zv = jnp.zeros((16,), jnp.float32)

            def zero_vec(i, _):
                rows[0, i // 8, pl.ds((i % 8) * 16, 16)] = zv
                return 0
            lax.fori_loop(0, K * d // 16, zero_vec, 0)
            nfull = rpt // K
            for q in range(nfull):
                pltpu.sync_copy(rows.at[0], accum.at[pl.ds(row0 + q * K, K)])
            rem = rpt - nfull * K
            if rem:
                pltpu.sync_copy(rows.at[0].at[pl.ds(0, rem)],
                                accum.at[pl.ds(row0 + nfull * K, rem)])

        plsc.subcore_barrier()

        def issue_pref(j, b):
            pltpu.async_copy(src_hbm.at[pl.ds(e_base + j * K, K)],
                             srci.at[b], dsem.at[b])
            pltpu.async_copy(dst_hbm.at[pl.ds(e_base + j * K, K)],
                             dsti.at[b], dsem.at[b])

        def issue_gather(j, b):
            pltpu.async_copy(h_hbm.at[srci.at[b]],
                             rows.at[b], gsem.at[b])

        def wait_pref(b):
            pltpu.make_async_copy(src_hbm.at[pl.ds(0, K)], srci.at[b],
                                  dsem.at[b]).wait()
            pltpu.make_async_copy(dst_hbm.at[pl.ds(0, K)], dsti.at[b],
                                  dsem.at[b]).wait()

        def wait_gather(b):
            pltpu.make_async_copy(h_hbm.at[pl.ds(0, K)], rows.at[b],
                                  gsem.at[b]).wait()

        def issue_scatter(b):
            pltpu.async_copy(rows.at[b], accum.at[dsti.at[b]], ssem.at[b],
                             add=True)

        def wait_scatter(b):
            pltpu.make_async_copy(rows.at[b], accum.at[pl.ds(0, K)],
                                  ssem.at[b]).wait()

        for j in range(3):
            issue_pref(j, j)
        for j in range(2):
            wait_pref(j)
            issue_gather(j, j)
        for j in range(3):
            wait_gather(j % NBUF)
            issue_scatter(j % NBUF)
            if j + 3 < NBUF:
                issue_pref(j + 3, j + 3)
            else:
                wait_scatter((j + 3) % NBUF)
                issue_pref(j + 3, (j + 3) % NBUF)
            wait_pref((j + 2) % NBUF)
            issue_gather(j + 2, (j + 2) % NBUF)

        def body(j, _):
            b = lax.rem(j, NBUF)
            bp = lax.rem(j + 3, NBUF)
            bg = lax.rem(j + 2, NBUF)
            wait_gather(b)
            issue_scatter(b)
            wait_scatter(bp)
            issue_pref(j + 3, bp)
            wait_pref(bg)
            issue_gather(j + 2, bg)
            return 0
        lax.fori_loop(3, n_chunks - 3, body, 0)

        j = n_chunks - 3
        wait_gather(j % NBUF)
        issue_scatter(j % NBUF)
        wait_pref((j + 2) % NBUF)
        issue_gather(j + 2, (j + 2) % NBUF)
        for j in range(n_chunks - 2, n_chunks):
            wait_gather(j % NBUF)
            issue_scatter(j % NBUF)
        for j in range(n_chunks - NBUF, n_chunks):
            wait_scatter(j % NBUF)

        plsc.subcore_barrier()

        pltpu.sync_copy(accum.at[pl.ds(row0, rpt)], part_hbm.at[c, pl.ds(row0, rpt)])

    return agg


def _mlp_block(p0_ref, p1_ref, w1_ref, b1_ref, w2_ref, b2_ref, o_ref):
    z = p0_ref[...] + p1_ref[...]
    h1 = jnp.dot(z, w1_ref[...], preferred_element_type=jnp.float32) + b1_ref[...]
    h1 = jnp.maximum(h1, 0.0)
    h2 = jnp.dot(h1, w2_ref[...], preferred_element_type=jnp.float32) + b2_ref[...]
    o_ref[...] = jnp.maximum(h2, 0.0)


def _make_tc_mlp(n, d, h_dim, block_rows):
    assert n % block_rows == 0
    grid = (n // block_rows,)
    row_spec = pl.BlockSpec((block_rows, d), lambda i: (i, 0))
    full = lambda r, c0: pl.BlockSpec((r, c0), lambda i: (0, 0))
    return pl.pallas_call(
        _mlp_block,
        grid=grid,
        in_specs=[row_spec, row_spec,
                  full(d, h_dim), full(1, h_dim),
                  full(h_dim, h_dim), full(1, h_dim)],
        out_specs=pl.BlockSpec((block_rows, h_dim), lambda i: (i, 0)),
        out_shape=jax.ShapeDtypeStruct((n, h_dim), jnp.float32),
    )


def kernel(x, edge_index, params):
    n, d = x.shape
    e = edge_index.shape[1]
    src = edge_index[0].astype(jnp.int32)
    dst = edge_index[1].astype(jnp.int32)

    agg = _make_sc_aggregate(n, d, e)
    mlp = _make_tc_mlp(n, d, d, 2000)

    inv = 1.0 / jnp.sqrt(1.0 + BN_EPS)
    h = x
    for i in range(3):
        s1 = params[f"g1_{i}"] * inv
        w1f = params[f"W1_{i}"] * s1[None, :]
        b1f = (params[f"b1_{i}"] * s1 + params[f"be1_{i}"])[None, :]
        s2 = params[f"g2_{i}"] * inv
        w2f = params[f"W2_{i}"] * s2[None, :]
        b2f = (params[f"b2_{i}"] * s2 + params[f"be2_{i}"])[None, :]
        part = agg(h, src, dst)
        h = mlp(part[0], part[1], w1f, b1f, w2f, b2f)
    return h
